# baseline (speedup 1.0000x reference)
import jax
import jax.numpy as jnp
from jax import lax
from jax.experimental import pallas as pl
from jax.experimental.pallas import tpu as pltpu
from jax.experimental.pallas import tpu_sc as plsc

VOCAB = 1000000
D_MODEL = 64
BATCH = 4096
SEQ = 200

_info = plsc.get_sparse_core_info()
NC, NS = _info.num_cores, _info.num_subcores
NW = NC * NS
L = 16

NFULL = VOCAB // 128
ATAIL = VOCAB - NFULL * 128
BLK_PER_W = NFULL // NW
NTAILW = NFULL - BLK_PER_W * NW

NBBLK = BATCH // 128


def _iota16(off):
    return jax.lax.iota(jnp.int32, 16) + off


def _transpose_kernel(wt_hbm, w2_hbm, tbuf, obuf, tailb, isem, osem):
    wid = lax.axis_index("s") * NC + lax.axis_index("c")
    rows_c = [_iota16(16 * (g % 4)) for g in range(8)]

    def start_in(b, k):
        pltpu.async_copy(wt_hbm.at[:, pl.ds(b * 128, 128)], tbuf.at[k],
                         isem.at[k])

    def wait_in(b, k):
        pltpu.make_async_copy(wt_hbm.at[:, pl.ds(b * 128, 128)], tbuf.at[k],
                              isem.at[k]).wait()

    def start_out(b, k):
        pltpu.async_copy(obuf.at[k], w2_hbm.at[pl.ds(b * 64, 64), :],
                         osem.at[k])

    def wait_out(b, k):
        pltpu.make_async_copy(obuf.at[k], w2_hbm.at[pl.ds(b * 64, 64), :],
                              osem.at[k]).wait()

    def transpose_into(tref, k, nrows):
        def row(r, carry):
            for g in range(8):
                col = jnp.full((16,), 2 * r + (1 if g >= 4 else 0), jnp.int32)
                v = plsc.load_gather(tref, [rows_c[g], col])
                obuf[k, r, pl.ds(g * 16, 16)] = v
            return carry
        lax.fori_loop(0, nrows, row, 0, unroll=2)

    base = wid * BLK_PER_W
    start_in(base, 0)
    start_in(base + 1, 1)

    def pair(p, carry):
        b0 = base + 2 * p
        for k in range(2):
            b = b0 + k
            wait_in(b, k)
            @pl.when(p > 0)
            def _():
                wait_out(b - 2, k)
            transpose_into(tbuf.at[k], k, 64)
            start_out(b, k)
            @pl.when(2 * p + k + 2 < BLK_PER_W)
            def _():
                start_in(b + 2, k)
        return carry

    lax.fori_loop(0, BLK_PER_W // 2, pair, 0)
    wait_out(base + BLK_PER_W - 2, 0)
    wait_out(base + BLK_PER_W - 1, 1)

    @pl.when(wid < NTAILW)
    def _():
        b = NW * BLK_PER_W + wid
        pltpu.async_copy(wt_hbm.at[:, pl.ds(b * 128, 128)], tbuf.at[0],
                         isem.at[0])
        pltpu.make_async_copy(wt_hbm.at[:, pl.ds(b * 128, 128)], tbuf.at[0],
                              isem.at[0]).wait()
        transpose_into(tbuf.at[0], 0, 64)
        pltpu.async_copy(obuf.at[0], w2_hbm.at[pl.ds(b * 64, 64), :],
                         osem.at[0])
        pltpu.make_async_copy(obuf.at[0], w2_hbm.at[pl.ds(b * 64, 64), :],
                              osem.at[0]).wait()

    @pl.when(wid == NTAILW)
    def _():
        b = NFULL
        src = wt_hbm.at[:, pl.ds(b * 128, ATAIL)]
        pltpu.async_copy(src, tailb, isem.at[0])
        pltpu.make_async_copy(src, tailb, isem.at[0]).wait()
        transpose_into(tailb, 0, ATAIL // 2)
        od = w2_hbm.at[pl.ds(b * 64, ATAIL // 2), :]
        osrc = obuf.at[0, pl.ds(0, ATAIL // 2), :]
        pltpu.async_copy(osrc, od, osem.at[0])
        pltpu.make_async_copy(osrc, od, osem.at[0]).wait()


def _gather_kernel(w2_hbm, tT_hbm, out_hbm, idxs, idx2, gbuf, obuf,
                   lsem, gsem, osem):
    wid = lax.axis_index("s") * NC + lax.axis_index("c")
    rows_g = [_iota16(16 * g) for g in range(8)]

    pltpu.async_copy(tT_hbm.at[:, pl.ds(wid * 128, 128)], idxs, lsem)
    pltpu.make_async_copy(tT_hbm.at[:, pl.ds(wid * 128, 128)], idxs,
                          lsem).wait()

    def prep(s, carry):
        for g in range(8):
            v = idxs[s, pl.ds(g * 16, 16)]
            idx2[s, pl.ds(g * 16, 16)] = lax.shift_right_logical(v, 1)
        return carry
    lax.fori_loop(0, SEQ, prep, 0, unroll=2)

    def start_gather(s, k):
        pltpu.async_copy(w2_hbm.at[idx2.at[s]], gbuf.at[k], gsem.at[k])

    def wait_gather(s, k):
        pltpu.make_async_copy(w2_hbm.at[idx2.at[s]], gbuf.at[k],
                              gsem.at[k]).wait()

    def start_out(s, k):
        pltpu.async_copy(obuf.at[k], out_hbm.at[s, :, pl.ds(wid * 128, 128)],
                         osem.at[k])

    def wait_out(s, k):
        pltpu.make_async_copy(obuf.at[k], out_hbm.at[s, :, pl.ds(wid * 128, 128)],
                              osem.at[k]).wait()

    def build(s, k):
        pars = []
        for g in range(8):
            v = idxs[s, pl.ds(g * 16, 16)]
            pars.append((v & 1) * 64)

        def col(c, carry):
            for g in range(8):
                v = plsc.load_gather(gbuf.at[k], [rows_g[g], pars[g] + c])
                obuf[k, c, pl.ds(g * 16, 16)] = v
            return carry
        lax.fori_loop(0, 64, col, 0, unroll=2)

    start_gather(0, 0)
    start_gather(1, 1)

    def pair(p, carry):
        s0 = 2 * p
        for k in range(2):
            s = s0 + k
            wait_gather(s, k)
            @pl.when(p > 0)
            def _():
                wait_out(s - 2, k)
            build(s, k)
            start_out(s, k)
            @pl.when(s + 2 < SEQ)
            def _():
                start_gather(s + 2, k)
        return carry

    lax.fori_loop(0, SEQ // 2, pair, 0)
    wait_out(SEQ - 2, 0)
    wait_out(SEQ - 1, 1)


@jax.jit
def _embed(token_ids, weight):
    wt = weight.T
    tT = token_ids.T
    mesh = plsc.VectorSubcoreMesh(core_axis_name="c", subcore_axis_name="s")
    w2 = pl.kernel(
        _transpose_kernel,
        mesh=mesh,
        out_type=jax.ShapeDtypeStruct((VOCAB // 2, 128), jnp.float32),
        scratch_types=[
            pltpu.VMEM((2, 64, 128), jnp.float32),
            pltpu.VMEM((2, 64, 128), jnp.float32),
            pltpu.VMEM((64, ATAIL), jnp.float32),
            pltpu.SemaphoreType.DMA((2,)),
            pltpu.SemaphoreType.DMA((2,)),
        ],
        compiler_params=pltpu.CompilerParams(use_tc_tiling_on_sc=True, needs_layout_passes=False),
    )(wt)
    outT = pl.kernel(
        _gather_kernel,
        mesh=mesh,
        out_type=jax.ShapeDtypeStruct((SEQ, D_MODEL, BATCH), jnp.float32),
        scratch_types=[
            pltpu.VMEM((SEQ, 128), jnp.int32),
            pltpu.VMEM((SEQ, 128), jnp.int32),
            pltpu.VMEM((2, 128, 128), jnp.float32),
            pltpu.VMEM((2, 64, 128), jnp.float32),
            pltpu.SemaphoreType.DMA,
            pltpu.SemaphoreType.DMA((2,)),
            pltpu.SemaphoreType.DMA((2,)),
        ],
        compiler_params=pltpu.CompilerParams(use_tc_tiling_on_sc=True, needs_layout_passes=False),
    )(w2, tT)
    return outT.transpose(2, 0, 1)


def kernel(token_ids, weight):
    return _embed(token_ids, weight)


# speedup vs baseline: 2.2156x; 2.2156x over previous
import jax
import jax.numpy as jnp
from jax import lax
from jax.experimental import pallas as pl
from jax.experimental.pallas import tpu as pltpu
from jax.experimental.pallas import tpu_sc as plsc

VOCAB = 1000000
D_MODEL = 64
BATCH = 4096
SEQ = 200

CHUNK = 128
NBUF = 8
N_TOKENS = BATCH * SEQ
_info = plsc.get_sparse_core_info()
NC, NS = _info.num_cores, _info.num_subcores
NW = NC * NS
CHUNKS_PER_W = N_TOKENS // (NW * CHUNK)
ROWS_PER_W = CHUNKS_PER_W * CHUNK
NGROUPS = CHUNKS_PER_W // NBUF


def _gather_kernel(w_hbm, idx_hbm, out_hbm, idx_v, rows_v, gsem, ssem):
    wid = lax.axis_index("s") * NC + lax.axis_index("c")
    pltpu.sync_copy(idx_hbm.at[pl.ds(wid * CHUNKS_PER_W, CHUNKS_PER_W)], idx_v)
    base = wid * ROWS_PER_W

    def start_gather(j, b):
        pltpu.async_copy(w_hbm.at[idx_v.at[j]], rows_v.at[b], gsem.at[b])

    def wait_gather(b):
        pltpu.make_async_copy(
            w_hbm.at[idx_v.at[0]], rows_v.at[b], gsem.at[b]).wait()

    def start_store(j, b):
        pltpu.async_copy(
            rows_v.at[b], out_hbm.at[pl.ds(base + j * CHUNK, CHUNK)],
            ssem.at[b])

    def wait_store(b):
        pltpu.make_async_copy(
            rows_v.at[b], out_hbm.at[pl.ds(base, CHUNK)], ssem.at[b]).wait()

    for b in range(NBUF):
        start_gather(b, b)

    def group(g, carry):
        for b in range(NBUF):
            wait_gather(b)
            start_store(g * NBUF + b, b)
        for b in range(NBUF):
            wait_store(b)
            start_gather((g + 1) * NBUF + b, b)
        return carry

    lax.fori_loop(0, NGROUPS - 1, group, 0)

    for b in range(NBUF):
        wait_gather(b)
        start_store((NGROUPS - 1) * NBUF + b, b)
    for b in range(NBUF):
        wait_store(b)


@jax.jit
def _embed(token_ids, weight):
    idx2d = token_ids.reshape(NW * CHUNKS_PER_W, CHUNK)
    mesh = plsc.VectorSubcoreMesh(core_axis_name="c", subcore_axis_name="s")
    out = pl.kernel(
        _gather_kernel,
        mesh=mesh,
        out_type=jax.ShapeDtypeStruct((N_TOKENS, D_MODEL), jnp.float32),
        scratch_types=[
            pltpu.VMEM((CHUNKS_PER_W, CHUNK), jnp.int32),
            pltpu.VMEM((NBUF, CHUNK, D_MODEL), jnp.float32),
            pltpu.SemaphoreType.DMA((NBUF,)),
            pltpu.SemaphoreType.DMA((NBUF,)),
        ],
        compiler_params=pltpu.CompilerParams(use_tc_tiling_on_sc=False),
    )(weight, idx2d)
    return out.reshape(BATCH, SEQ, D_MODEL)


def kernel(token_ids, weight):
    return _embed(token_ids, weight)


# speedup vs baseline: 2.2187x; 1.0014x over previous
import jax
import jax.numpy as jnp
from jax import lax
from jax.experimental import pallas as pl
from jax.experimental.pallas import tpu as pltpu
from jax.experimental.pallas import tpu_sc as plsc

VOCAB = 1000000
D_MODEL = 64
BATCH = 4096
SEQ = 200

CHUNK = 128
NBUF = 10
N_TOKENS = BATCH * SEQ
_info = plsc.get_sparse_core_info()
NC, NS = _info.num_cores, _info.num_subcores
NW = NC * NS
CHUNKS_PER_W = N_TOKENS // (NW * CHUNK)
ROWS_PER_W = CHUNKS_PER_W * CHUNK
NGROUPS = CHUNKS_PER_W // NBUF


def _gather_kernel(w_hbm, idx_hbm, out_hbm, idx_v, rows_v, gsem, ssem):
    wid = lax.axis_index("s") * NC + lax.axis_index("c")
    pltpu.sync_copy(idx_hbm.at[pl.ds(wid * CHUNKS_PER_W, CHUNKS_PER_W)], idx_v)
    base = wid * ROWS_PER_W

    def start_gather(j, b):
        pltpu.async_copy(w_hbm.at[idx_v.at[j]], rows_v.at[b], gsem.at[b])

    def wait_gather(b):
        pltpu.make_async_copy(
            w_hbm.at[idx_v.at[0]], rows_v.at[b], gsem.at[b]).wait()

    def start_store(j, b):
        pltpu.async_copy(
            rows_v.at[b], out_hbm.at[pl.ds(base + j * CHUNK, CHUNK)],
            ssem.at[b])

    def wait_store(b):
        pltpu.make_async_copy(
            rows_v.at[b], out_hbm.at[pl.ds(base, CHUNK)], ssem.at[b]).wait()

    for b in range(NBUF):
        start_gather(b, b)

    def group(g, carry):
        for b in range(NBUF):
            wait_gather(b)
            start_store(g * NBUF + b, b)
        for b in range(NBUF):
            wait_store(b)
            start_gather((g + 1) * NBUF + b, b)
        return carry

    lax.fori_loop(0, NGROUPS - 1, group, 0)

    for b in range(NBUF):
        wait_gather(b)
        start_store((NGROUPS - 1) * NBUF + b, b)
    for b in range(NBUF):
        wait_store(b)


@jax.jit
def _embed(token_ids, weight):
    idx2d = token_ids.reshape(NW * CHUNKS_PER_W, CHUNK)
    mesh = plsc.VectorSubcoreMesh(core_axis_name="c", subcore_axis_name="s")
    out = pl.kernel(
        _gather_kernel,
        mesh=mesh,
        out_type=jax.ShapeDtypeStruct((N_TOKENS, D_MODEL), jnp.float32),
        scratch_types=[
            pltpu.VMEM((CHUNKS_PER_W, CHUNK), jnp.int32),
            pltpu.VMEM((NBUF, CHUNK, D_MODEL), jnp.float32),
            pltpu.SemaphoreType.DMA((NBUF,)),
            pltpu.SemaphoreType.DMA((NBUF,)),
        ],
        compiler_params=pltpu.CompilerParams(use_tc_tiling_on_sc=False),
    )(weight, idx2d)
    return out.reshape(BATCH, SEQ, D_MODEL)


def kernel(token_ids, weight):
    return _embed(token_ids, weight)
